# baseline (speedup 1.0000x reference)
import functools

import jax
import jax.numpy as jnp
from jax import lax
from jax.experimental import pallas as pl
from jax.experimental.pallas import tpu as pltpu
from jax.experimental.pallas import tpu_sc as plsc

S, H, FF, E = 2048, 1024, 2816, 8
T = 256
NT = 24
NPAD = NT * T
NP = 2 * S
NFC = 2
FFC = FF // NFC
ST = S // T


def _vgather16(v, idx):
    dn = lax.GatherDimensionNumbers(offset_dims=(), collapsed_slice_dims=(0,),
                                    start_index_map=(0,))
    return lax.gather(v, idx[:, None], dn, slice_sizes=(1,),
                      mode=lax.GatherScatterMode.PROMISE_IN_BOUNDS)


def _router_body(x_ref, rw_ref, rb_ref, eb_ref, i1_ref, i2_ref, g1_ref, g2_ref):
    x = x_ref[...]
    logits = lax.dot_general(x, rw_ref[...], (((1,), (1,)), ((), ())),
                             preferred_element_type=jnp.float32)
    logits = logits + rb_ref[...][None, :]
    lb = logits + eb_ref[...][None, :]
    eio = lax.broadcasted_iota(jnp.int32, lb.shape, 1)
    big = jnp.int32(1 << 30)
    m1 = jnp.max(lb, axis=-1, keepdims=True)
    i1 = jnp.min(jnp.where(lb == m1, eio, big), axis=-1, keepdims=True)
    lb2 = jnp.where(eio == i1, -jnp.inf, lb)
    m2 = jnp.max(lb2, axis=-1, keepdims=True)
    i2 = jnp.min(jnp.where(lb2 == m2, eio, big), axis=-1, keepdims=True)
    s1 = jnp.sum(jnp.where(eio == i1, logits, 0.0), axis=-1)
    s2 = jnp.sum(jnp.where(eio == i2, logits, 0.0), axis=-1)
    p1 = jax.nn.sigmoid(s1)
    p2 = jax.nn.sigmoid(s2)
    den = jnp.maximum(p1 + p2, 1e-12)
    i1_ref[...] = i1[:, 0]
    i2_ref[...] = i2[:, 0]
    g1_ref[...] = p1 / den
    g2_ref[...] = p2 / den


def _router(x2d, router_w, router_b, expert_biases):
    return pl.pallas_call(
        _router_body,
        out_shape=[
            jax.ShapeDtypeStruct((S,), jnp.int32),
            jax.ShapeDtypeStruct((S,), jnp.int32),
            jax.ShapeDtypeStruct((S,), jnp.float32),
            jax.ShapeDtypeStruct((S,), jnp.float32),
        ],
    )(x2d, router_w, router_b, expert_biases)


def _dispatch(i1, i2, g1, g2, x2d):
    mesh = plsc.VectorSubcoreMesh(core_axis_name="c", subcore_axis_name="s")
    out_type = [
        jax.ShapeDtypeStruct((NPAD, H), jnp.float32),
        jax.ShapeDtypeStruct((NPAD,), jnp.float32),
        jax.ShapeDtypeStruct((32,), jnp.int32),
        jax.ShapeDtypeStruct((S,), jnp.int32),
        jax.ShapeDtypeStruct((S,), jnp.int32),
    ]
    DS = NPAD // 16
    DP = NP // 16
    scratch = [
        pltpu.VMEM((NP,), jnp.int32),
        pltpu.VMEM((NP,), jnp.float32),
        pltpu.VMEM((NPAD,), jnp.int32),
        pltpu.VMEM((NPAD,), jnp.float32),
        pltpu.VMEM((NP,), jnp.int32),
        pltpu.VMEM((16,), jnp.int32),
        pltpu.VMEM((128,), jnp.int32),
        pltpu.VMEM((8, DS), jnp.int32),
        pltpu.VMEM((8, DS), jnp.float32),
        pltpu.VMEM((8, DP), jnp.int32),
        pltpu.VMEM((DS,), jnp.int32),
        pltpu.VMEM((DS,), jnp.float32),
        pltpu.VMEM((DP,), jnp.int32),
        pltpu.VMEM((32,), jnp.int32),
        pltpu.VMEM_SHARED((128,), jnp.int32),
        pltpu.VMEM_SHARED((8, NPAD), jnp.int32),
        pltpu.VMEM_SHARED((8, NPAD), jnp.float32),
        pltpu.VMEM_SHARED((8, NP), jnp.int32),
        pltpu.VMEM_SHARED((NPAD,), jnp.int32),
        pltpu.VMEM((64,), jnp.int32),
        pltpu.VMEM((64, H), jnp.float32),
        pltpu.SemaphoreType.DMA,
    ]

    @functools.partial(
        pl.kernel, mesh=mesh, out_type=out_type, scratch_types=scratch,
        compiler_params=pltpu.CompilerParams(needs_layout_passes=False))
    def body(i1h, i2h, g1h, g2h, xh, xs, gs, te, pos1, pos2,
             ep, gp, ptokp, gsortp, posbp, st16, cnta, mpt, mgs, mpb,
             mmpt, mmgs, mmpb, tebuf, sh_cnt, sh_ptokp, sh_gsortp, sh_posbp,
             shp, myidx, rows, sem):
        c = lax.axis_index("c")
        s = lax.axis_index("s")
        io16 = lax.iota(jnp.int32, 16)
        zi = jnp.zeros((16,), jnp.int32)
        zf = jnp.zeros((16,), jnp.float32)

        @pl.when(s < E)
        def _count():
            pltpu.sync_copy(i1h, ep.at[pl.ds(0, S)])
            pltpu.sync_copy(i2h, ep.at[pl.ds(S, S)])
            pltpu.sync_copy(g1h, gp.at[pl.ds(0, S)])
            pltpu.sync_copy(g2h, gp.at[pl.ds(S, S)])

            def init_body(i, carry):
                ptokp[pl.ds(i * 16, 16)] = zi
                gsortp[pl.ds(i * 16, 16)] = zf
                return carry

            lax.fori_loop(0, NPAD // 16, init_body, 0)

            def cbody(g, cnt):
                ev = ep[pl.ds(g * 16, 16)]
                return cnt + plsc.all_reduce_population_count(ev == s)

            cnt = lax.fori_loop(0, NP // 16, cbody, zi)
            st16[...] = cnt
            pltpu.sync_copy(st16, sh_cnt.at[pl.ds(s * 16, 16)])

        plsc.subcore_barrier()

        pltpu.sync_copy(sh_cnt, cnta)
        counts_v = zi
        for e in range(E):
            sp = cnta[pl.ds(e * 16, 16)]
            counts_v = jnp.where(io16 == e, sp, counts_v)
        ntiles_v = (counts_v + (T - 1)) // T
        tstart_v = plsc.cumsum(ntiles_v) - ntiles_v
        basevals = tstart_v * T

        @pl.when(s < E)
        def _scatter():
            mybase = _vgather16(basevals, jnp.full((16,), 0, jnp.int32) + s)

            def group2(g, carry):
                pidx, crun = carry
                ev = ep[pl.ds(g * 16, 16)]
                gv = gp[pl.ds(g * 16, 16)]
                m = ev == s
                mi = m.astype(jnp.int32)
                incl = plsc.cumsum(mi)
                posv = jnp.where(m, mybase + incl - 1 + crun, 0)
                tokv = pidx & (S - 1)
                plsc.store_scatter(ptokp, [posv], tokv, mask=m)
                plsc.store_scatter(gsortp, [posv], gv, mask=m)
                posbp[pl.ds(g * 16, 16)] = posv
                tote = plsc.cummax(lax.rev(incl, (0,)))
                return (pidx + 16, crun + tote)

            lax.fori_loop(0, NP // 16, group2, (io16, zi))
            pltpu.sync_copy(ptokp, sh_ptokp.at[s])
            pltpu.sync_copy(gsortp, sh_gsortp.at[s])
            pltpu.sync_copy(posbp, sh_posbp.at[s])

        @pl.when((s == 0) & (c == 0))
        def _te():
            for j in range(2):
                tv = io16 + j * 16
                ind = jnp.zeros((16,), jnp.int32)
                for e in range(E):
                    tse = _vgather16(tstart_v, jnp.full((16,), e, jnp.int32))
                    ind = ind + jnp.where(tv >= tse, 1, 0)
                tebuf[pl.ds(j * 16, 16)] = jnp.maximum(ind - 1, 0)
            pltpu.sync_copy(tebuf, te)

        plsc.subcore_barrier()

        DS = NPAD // 16
        DP = NP // 16
        pltpu.sync_copy(sh_ptokp.at[:, pl.ds(s * DS, DS)], mpt)
        pltpu.sync_copy(sh_gsortp.at[:, pl.ds(s * DS, DS)], mgs)
        pltpu.sync_copy(sh_posbp.at[:, pl.ds(s * DP, DP)], mpb)

        def merge_s(k, carry):
            acc_i = zi
            acc_f = zf
            for e in range(E):
                acc_i = acc_i + mpt[e, pl.ds(k * 16, 16)]
                acc_f = acc_f + mgs[e, pl.ds(k * 16, 16)]
            mmpt[pl.ds(k * 16, 16)] = acc_i
            mmgs[pl.ds(k * 16, 16)] = acc_f
            return carry

        lax.fori_loop(0, DS // 16, merge_s, 0)

        def merge_p(k, carry):
            acc_i = zi
            for e in range(E):
                acc_i = acc_i + mpb[e, pl.ds(k * 16, 16)]
            mmpb[pl.ds(k * 16, 16)] = acc_i
            return carry

        lax.fori_loop(0, DP // 16, merge_p, 0)

        pltpu.sync_copy(mmpt, shp.at[pl.ds(s * DS, DS)])

        @pl.when(c == 0)
        def _meta():
            pltpu.sync_copy(mmgs, gs.at[pl.ds(s * DS, DS)])

            @pl.when(s < 8)
            def _p1():
                pltpu.sync_copy(mmpb, pos1.at[pl.ds(s * DP, DP)])

            @pl.when(s >= 8)
            def _p2():
                pltpu.sync_copy(mmpb, pos2.at[pl.ds((s - 8) * DP, DP)])

        plsc.subcore_barrier()

        half = NPAD // 2
        per = half // 16
        for ch in range(per // 64):
            start = c * half + s * per + ch * 64
            pltpu.sync_copy(shp.at[pl.ds(start, 64)], myidx)
            pltpu.async_copy(xh.at[myidx], rows, sem).wait()
            pltpu.sync_copy(rows, xs.at[pl.ds(start, 64)])

    return body(i1, i2, g1, g2, x2d)


def _ffn_routed_body(te_ref, xs_ref, g_ref, w1_ref, b1_ref, w2_ref, b2_ref,
                     w3_ref, b3_ref, out_ref):
    j = pl.program_id(1)
    xt = xs_ref[...].astype(jnp.bfloat16)
    gcol = g_ref[0, 0][:, None]
    h1 = lax.dot_general(xt, w1_ref[0].astype(jnp.bfloat16),
                         (((1,), (1,)), ((), ())),
                         preferred_element_type=jnp.float32) + b1_ref[0, 0]
    h3 = lax.dot_general(xt, w3_ref[0].astype(jnp.bfloat16),
                         (((1,), (1,)), ((), ())),
                         preferred_element_type=jnp.float32) + b3_ref[0, 0]
    hh = (h1 * jax.nn.sigmoid(h1) * h3).astype(jnp.bfloat16)
    y = lax.dot_general(hh, w2_ref[0].astype(jnp.bfloat16),
                        (((1,), (1,)), ((), ())),
                        preferred_element_type=jnp.float32)

    @pl.when(j == 0)
    def _():
        out_ref[...] = (y + b2_ref[0]) * gcol

    @pl.when(j != 0)
    def _():
        out_ref[...] = out_ref[...] + y * gcol


def _ffn_routed(te, xs, gs3, rw1, rb1, rw2, rb2, rw3, rb3):
    grid_spec = pltpu.PrefetchScalarGridSpec(
        num_scalar_prefetch=1,
        grid=(NT, NFC),
        in_specs=[
            pl.BlockSpec((T, H), lambda i, j, te: (i, 0)),
            pl.BlockSpec((1, 1, T), lambda i, j, te: (i, 0, 0)),
            pl.BlockSpec((1, FFC, H), lambda i, j, te: (te[i], j, 0)),
            pl.BlockSpec((1, 1, 1, FFC), lambda i, j, te: (te[i], j, 0, 0)),
            pl.BlockSpec((1, H, FFC), lambda i, j, te: (te[i], 0, j)),
            pl.BlockSpec((1, 1, H), lambda i, j, te: (te[i], 0, 0)),
            pl.BlockSpec((1, FFC, H), lambda i, j, te: (te[i], j, 0)),
            pl.BlockSpec((1, 1, 1, FFC), lambda i, j, te: (te[i], j, 0, 0)),
        ],
        out_specs=pl.BlockSpec((T, H), lambda i, j, te: (i, 0)),
    )
    return pl.pallas_call(
        _ffn_routed_body,
        grid_spec=grid_spec,
        out_shape=jax.ShapeDtypeStruct((NPAD, H), jnp.float32),
        compiler_params=pltpu.CompilerParams(
            dimension_semantics=("arbitrary", "arbitrary"),
            vmem_limit_bytes=128 * 1024 * 1024,
        ),
    )(te, xs, gs3, rw1, rb1, rw2, rb2, rw3, rb3)


def _combine_gather(pos1, pos2, yr):
    mesh = plsc.VectorSubcoreMesh(core_axis_name="c", subcore_axis_name="s")
    out_type = [
        jax.ShapeDtypeStruct((S, H), jnp.float32),
        jax.ShapeDtypeStruct((S, H), jnp.float32),
    ]
    scratch = [
        pltpu.VMEM((64,), jnp.int32),
        pltpu.VMEM((64, H), jnp.float32),
        pltpu.SemaphoreType.DMA,
    ]

    @functools.partial(pl.kernel, mesh=mesh, out_type=out_type,
                       scratch_types=scratch)
    def body(p1h, p2h, yh, a, b, myidx, rows, sem):
        c = lax.axis_index("c")
        s = lax.axis_index("s")
        base = (c * 16 + s) * 64
        pltpu.sync_copy(p1h.at[pl.ds(base, 64)], myidx)
        pltpu.async_copy(yh.at[myidx], rows, sem).wait()
        pltpu.sync_copy(rows, a.at[pl.ds(base, 64)])
        pltpu.sync_copy(p2h.at[pl.ds(base, 64)], myidx)
        pltpu.async_copy(yh.at[myidx], rows, sem).wait()
        pltpu.sync_copy(rows, b.at[pl.ds(base, 64)])

    return body(pos1, pos2, yr)


def _ffn_shared_body(x_ref, w1_ref, b1_ref, w2_ref, b2_ref, w3_ref, b3_ref,
                     a_ref, b2r_ref, out_ref):
    j = pl.program_id(1)
    xt = x_ref[...].astype(jnp.bfloat16)
    h1 = lax.dot_general(xt, w1_ref[...].astype(jnp.bfloat16),
                         (((1,), (1,)), ((), ())),
                         preferred_element_type=jnp.float32) + b1_ref[0]
    h3 = lax.dot_general(xt, w3_ref[...].astype(jnp.bfloat16),
                         (((1,), (1,)), ((), ())),
                         preferred_element_type=jnp.float32) + b3_ref[0]
    hh = (h1 * jax.nn.sigmoid(h1) * h3).astype(jnp.bfloat16)
    y = lax.dot_general(hh, w2_ref[...].astype(jnp.bfloat16),
                        (((1,), (1,)), ((), ())),
                        preferred_element_type=jnp.float32)

    @pl.when(j == 0)
    def _():
        out_ref[...] = y + b2_ref[...][None, :] + a_ref[...] + b2r_ref[...]

    @pl.when(j != 0)
    def _():
        out_ref[...] = out_ref[...] + y


def _ffn_shared(x2d, sw1, sb1, sw2, sb2, sw3, sb3, a, b):
    return pl.pallas_call(
        _ffn_shared_body,
        grid=(ST, NFC),
        in_specs=[
            pl.BlockSpec((T, H), lambda i, j: (i, 0)),
            pl.BlockSpec((FFC, H), lambda i, j: (j, 0)),
            pl.BlockSpec((1, 1, FFC), lambda i, j: (j, 0, 0)),
            pl.BlockSpec((H, FFC), lambda i, j: (0, j)),
            pl.BlockSpec((H,), lambda i, j: (0,)),
            pl.BlockSpec((FFC, H), lambda i, j: (j, 0)),
            pl.BlockSpec((1, 1, FFC), lambda i, j: (j, 0, 0)),
            pl.BlockSpec((T, H), lambda i, j: (i, 0)),
            pl.BlockSpec((T, H), lambda i, j: (i, 0)),
        ],
        out_specs=pl.BlockSpec((T, H), lambda i, j: (i, 0)),
        out_shape=jax.ShapeDtypeStruct((S, H), jnp.float32),
        compiler_params=pltpu.CompilerParams(
            dimension_semantics=("arbitrary", "arbitrary"),
            vmem_limit_bytes=128 * 1024 * 1024,
        ),
    )(x2d, sw1, sb1, sw2, sb2, sw3, sb3, a, b)


def kernel(x, router_w, router_b, expert_biases, sw1, sb1, sw2, sb2, sw3, sb3,
           rw1, rb1, rw2, rb2, rw3, rb3):
    x2d = x.reshape(S, H)
    i1, i2, g1, g2 = _router(x2d, router_w, router_b, expert_biases)
    xs, gs, te, pos1, pos2 = _dispatch(i1, i2, g1, g2, x2d)
    gs3 = gs.reshape(NT, 1, T)
    rb1r = rb1.reshape(E, NFC, 1, FFC)
    rb3r = rb3.reshape(E, NFC, 1, FFC)
    rb2r = rb2.reshape(E, 1, H)
    yr = _ffn_routed(te, xs, gs3, rw1, rb1r, rw2, rb2r, rw3, rb3r)
    a, b = _combine_gather(pos1, pos2, yr)
    sb1r = sb1.reshape(NFC, 1, FFC)
    sb3r = sb3.reshape(NFC, 1, FFC)
    out = _ffn_shared(x2d, sw1, sb1r, sw2, sb2, sw3, sb3r, a, b)
    return out.reshape(1, S, H)


# speedup vs baseline: 1.1103x; 1.1103x over previous
import functools

import jax
import jax.numpy as jnp
from jax import lax
from jax.experimental import pallas as pl
from jax.experimental.pallas import tpu as pltpu
from jax.experimental.pallas import tpu_sc as plsc

S, H, FF, E = 2048, 1024, 2816, 8
T = 256
NT = 24
NPAD = NT * T
NP = 2 * S
NFC = 2
FFC = FF // NFC
ST = S // T


def _vgather16(v, idx):
    dn = lax.GatherDimensionNumbers(offset_dims=(), collapsed_slice_dims=(0,),
                                    start_index_map=(0,))
    return lax.gather(v, idx[:, None], dn, slice_sizes=(1,),
                      mode=lax.GatherScatterMode.PROMISE_IN_BOUNDS)


def _router_body(x_ref, rw_ref, rb_ref, eb_ref, i1_ref, i2_ref, g1_ref, g2_ref):
    x = x_ref[...]
    logits = lax.dot_general(x, rw_ref[...], (((1,), (1,)), ((), ())),
                             preferred_element_type=jnp.float32)
    logits = logits + rb_ref[...][None, :]
    lb = logits + eb_ref[...][None, :]
    eio = lax.broadcasted_iota(jnp.int32, lb.shape, 1)
    big = jnp.int32(1 << 30)
    m1 = jnp.max(lb, axis=-1, keepdims=True)
    i1 = jnp.min(jnp.where(lb == m1, eio, big), axis=-1, keepdims=True)
    lb2 = jnp.where(eio == i1, -jnp.inf, lb)
    m2 = jnp.max(lb2, axis=-1, keepdims=True)
    i2 = jnp.min(jnp.where(lb2 == m2, eio, big), axis=-1, keepdims=True)
    s1 = jnp.sum(jnp.where(eio == i1, logits, 0.0), axis=-1)
    s2 = jnp.sum(jnp.where(eio == i2, logits, 0.0), axis=-1)
    p1 = jax.nn.sigmoid(s1)
    p2 = jax.nn.sigmoid(s2)
    den = jnp.maximum(p1 + p2, 1e-12)
    i1_ref[...] = i1[:, 0]
    i2_ref[...] = i2[:, 0]
    g1_ref[...] = p1 / den
    g2_ref[...] = p2 / den


def _router(x2d, router_w, router_b, expert_biases):
    return pl.pallas_call(
        _router_body,
        out_shape=[
            jax.ShapeDtypeStruct((S,), jnp.int32),
            jax.ShapeDtypeStruct((S,), jnp.int32),
            jax.ShapeDtypeStruct((S,), jnp.float32),
            jax.ShapeDtypeStruct((S,), jnp.float32),
        ],
    )(x2d, router_w, router_b, expert_biases)


def _dispatch(i1, i2, g1, g2, x2d):
    mesh = plsc.VectorSubcoreMesh(core_axis_name="c", subcore_axis_name="s")
    out_type = [
        jax.ShapeDtypeStruct((NPAD, H), jnp.float32),
        jax.ShapeDtypeStruct((NPAD,), jnp.float32),
        jax.ShapeDtypeStruct((32,), jnp.int32),
        jax.ShapeDtypeStruct((S,), jnp.int32),
        jax.ShapeDtypeStruct((S,), jnp.int32),
    ]
    DS = NPAD // 16
    DP = NP // 16
    scratch = [
        pltpu.VMEM((NP,), jnp.int32),
        pltpu.VMEM((NP,), jnp.float32),
        pltpu.VMEM((NPAD,), jnp.int32),
        pltpu.VMEM((NPAD,), jnp.float32),
        pltpu.VMEM((NP,), jnp.int32),
        pltpu.VMEM((16,), jnp.int32),
        pltpu.VMEM((128,), jnp.int32),
        pltpu.VMEM((8, DS), jnp.int32),
        pltpu.VMEM((8, DS), jnp.float32),
        pltpu.VMEM((8, DP), jnp.int32),
        pltpu.VMEM((DS,), jnp.int32),
        pltpu.VMEM((DS,), jnp.float32),
        pltpu.VMEM((DP,), jnp.int32),
        pltpu.VMEM((32,), jnp.int32),
        pltpu.VMEM_SHARED((128,), jnp.int32),
        pltpu.VMEM_SHARED((8, NPAD), jnp.int32),
        pltpu.VMEM_SHARED((8, NPAD), jnp.float32),
        pltpu.VMEM_SHARED((8, NP), jnp.int32),
        pltpu.VMEM_SHARED((NPAD,), jnp.int32),
        pltpu.VMEM((64,), jnp.int32),
        pltpu.VMEM((64, H), jnp.float32),
        pltpu.SemaphoreType.DMA,
    ]

    @functools.partial(
        pl.kernel, mesh=mesh, out_type=out_type, scratch_types=scratch,
        compiler_params=pltpu.CompilerParams(needs_layout_passes=False))
    def body(i1h, i2h, g1h, g2h, xh, xs, gs, te, pos1, pos2,
             ep, gp, ptokp, gsortp, posbp, st16, cnta, mpt, mgs, mpb,
             mmpt, mmgs, mmpb, tebuf, sh_cnt, sh_ptokp, sh_gsortp, sh_posbp,
             shp, myidx, rows, sem):
        c = lax.axis_index("c")
        s = lax.axis_index("s")
        io16 = lax.iota(jnp.int32, 16)
        zi = jnp.zeros((16,), jnp.int32)
        zf = jnp.zeros((16,), jnp.float32)

        @pl.when(s < E)
        def _count():
            pltpu.sync_copy(i1h, ep.at[pl.ds(0, S)])
            pltpu.sync_copy(i2h, ep.at[pl.ds(S, S)])
            pltpu.sync_copy(g1h, gp.at[pl.ds(0, S)])
            pltpu.sync_copy(g2h, gp.at[pl.ds(S, S)])

            def init_body(i, carry):
                ptokp[pl.ds(i * 16, 16)] = zi
                gsortp[pl.ds(i * 16, 16)] = zf
                return carry

            lax.fori_loop(0, NPAD // 16, init_body, 0)

            def cbody(g, cnt):
                ev = ep[pl.ds(g * 16, 16)]
                return cnt + plsc.all_reduce_population_count(ev == s)

            cnt = lax.fori_loop(0, NP // 16, cbody, zi)
            st16[...] = cnt
            pltpu.sync_copy(st16, sh_cnt.at[pl.ds(s * 16, 16)])

        plsc.subcore_barrier()

        pltpu.sync_copy(sh_cnt, cnta)
        counts_v = zi
        for e in range(E):
            sp = cnta[pl.ds(e * 16, 16)]
            counts_v = jnp.where(io16 == e, sp, counts_v)
        ntiles_v = (counts_v + (T - 1)) // T
        tstart_v = plsc.cumsum(ntiles_v) - ntiles_v
        basevals = tstart_v * T

        @pl.when(s < E)
        def _scatter():
            mybase = _vgather16(basevals, jnp.full((16,), 0, jnp.int32) + s)

            def group2(g, carry):
                pidx, crun = carry
                ev = ep[pl.ds(g * 16, 16)]
                gv = gp[pl.ds(g * 16, 16)]
                m = ev == s
                mi = m.astype(jnp.int32)
                incl = plsc.cumsum(mi)
                posv = jnp.where(m, mybase + incl - 1 + crun, 0)
                tokv = pidx & (S - 1)
                plsc.store_scatter(ptokp, [posv], tokv, mask=m)
                plsc.store_scatter(gsortp, [posv], gv, mask=m)
                posbp[pl.ds(g * 16, 16)] = posv
                tote = plsc.cummax(lax.rev(incl, (0,)))
                return (pidx + 16, crun + tote)

            lax.fori_loop(0, NP // 16, group2, (io16, zi))
            pltpu.sync_copy(ptokp, sh_ptokp.at[s])
            pltpu.sync_copy(gsortp, sh_gsortp.at[s])
            pltpu.sync_copy(posbp, sh_posbp.at[s])

        @pl.when((s == 0) & (c == 0))
        def _te():
            for j in range(2):
                tv = io16 + j * 16
                ind = jnp.zeros((16,), jnp.int32)
                for e in range(E):
                    tse = _vgather16(tstart_v, jnp.full((16,), e, jnp.int32))
                    ind = ind + jnp.where(tv >= tse, 1, 0)
                tebuf[pl.ds(j * 16, 16)] = jnp.maximum(ind - 1, 0)
            pltpu.sync_copy(tebuf, te)

        plsc.subcore_barrier()

        DS = NPAD // 16
        DP = NP // 16
        pltpu.sync_copy(sh_ptokp.at[:, pl.ds(s * DS, DS)], mpt)
        pltpu.sync_copy(sh_gsortp.at[:, pl.ds(s * DS, DS)], mgs)
        pltpu.sync_copy(sh_posbp.at[:, pl.ds(s * DP, DP)], mpb)

        def merge_s(k, carry):
            acc_i = zi
            acc_f = zf
            for e in range(E):
                acc_i = acc_i + mpt[e, pl.ds(k * 16, 16)]
                acc_f = acc_f + mgs[e, pl.ds(k * 16, 16)]
            mmpt[pl.ds(k * 16, 16)] = acc_i
            mmgs[pl.ds(k * 16, 16)] = acc_f
            return carry

        lax.fori_loop(0, DS // 16, merge_s, 0)

        def merge_p(k, carry):
            acc_i = zi
            for e in range(E):
                acc_i = acc_i + mpb[e, pl.ds(k * 16, 16)]
            mmpb[pl.ds(k * 16, 16)] = acc_i
            return carry

        lax.fori_loop(0, DP // 16, merge_p, 0)

        pltpu.sync_copy(mmpt, shp.at[pl.ds(s * DS, DS)])

        @pl.when(c == 0)
        def _meta():
            pltpu.sync_copy(mmgs, gs.at[pl.ds(s * DS, DS)])

            @pl.when(s < 8)
            def _p1():
                pltpu.sync_copy(mmpb, pos1.at[pl.ds(s * DP, DP)])

            @pl.when(s >= 8)
            def _p2():
                pltpu.sync_copy(mmpb, pos2.at[pl.ds((s - 8) * DP, DP)])

        plsc.subcore_barrier()

        half = NPAD // 2
        per = half // 16
        for ch in range(per // 64):
            start = c * half + s * per + ch * 64
            pltpu.sync_copy(shp.at[pl.ds(start, 64)], myidx)
            pltpu.async_copy(xh.at[myidx], rows, sem).wait()
            pltpu.sync_copy(rows, xs.at[pl.ds(start, 64)])

    return body(i1, i2, g1, g2, x2d)


def _ffn_routed_body(te_ref, xs_ref, g_ref, w1_ref, b1_ref, w2_ref, b2_ref,
                     w3_ref, b3_ref, out_ref):
    j = pl.program_id(1)
    xt = xs_ref[...].astype(jnp.bfloat16)
    gcol = g_ref[0, 0][:, None]
    h1 = lax.dot_general(xt, w1_ref[0].astype(jnp.bfloat16),
                         (((1,), (1,)), ((), ())),
                         preferred_element_type=jnp.float32) + b1_ref[0, 0]
    h3 = lax.dot_general(xt, w3_ref[0].astype(jnp.bfloat16),
                         (((1,), (1,)), ((), ())),
                         preferred_element_type=jnp.float32) + b3_ref[0, 0]
    hh = (h1 * jax.nn.sigmoid(h1) * h3).astype(jnp.bfloat16)
    w2c = w2_ref[0, :, pl.ds(j * FFC, FFC)]
    y = lax.dot_general(hh, w2c.astype(jnp.bfloat16),
                        (((1,), (1,)), ((), ())),
                        preferred_element_type=jnp.float32)

    @pl.when(j == 0)
    def _():
        out_ref[...] = (y + b2_ref[0]) * gcol

    @pl.when(j != 0)
    def _():
        out_ref[...] = out_ref[...] + y * gcol


def _ffn_routed(te, xs, gs3, rw1, rb1, rw2, rb2, rw3, rb3):
    grid_spec = pltpu.PrefetchScalarGridSpec(
        num_scalar_prefetch=1,
        grid=(NT, NFC),
        in_specs=[
            pl.BlockSpec((T, H), lambda i, j, te: (i, 0)),
            pl.BlockSpec((1, 1, T), lambda i, j, te: (i, 0, 0)),
            pl.BlockSpec((1, FFC, H), lambda i, j, te: (te[i], j, 0)),
            pl.BlockSpec((1, 1, 1, FFC), lambda i, j, te: (te[i], j, 0, 0)),
            pl.BlockSpec((1, H, FF), lambda i, j, te: (te[i], 0, 0)),
            pl.BlockSpec((1, 1, H), lambda i, j, te: (te[i], 0, 0)),
            pl.BlockSpec((1, FFC, H), lambda i, j, te: (te[i], j, 0)),
            pl.BlockSpec((1, 1, 1, FFC), lambda i, j, te: (te[i], j, 0, 0)),
        ],
        out_specs=pl.BlockSpec((T, H), lambda i, j, te: (i, 0)),
    )
    return pl.pallas_call(
        _ffn_routed_body,
        grid_spec=grid_spec,
        out_shape=jax.ShapeDtypeStruct((NPAD, H), jnp.float32),
        compiler_params=pltpu.CompilerParams(
            dimension_semantics=("arbitrary", "arbitrary"),
            vmem_limit_bytes=128 * 1024 * 1024,
        ),
    )(te, xs, gs3, rw1, rb1, rw2, rb2, rw3, rb3)


def _combine_gather(pos1, pos2, yr):
    mesh = plsc.VectorSubcoreMesh(core_axis_name="c", subcore_axis_name="s")
    out_type = [
        jax.ShapeDtypeStruct((S, H), jnp.float32),
        jax.ShapeDtypeStruct((S, H), jnp.float32),
    ]
    scratch = [
        pltpu.VMEM((64,), jnp.int32),
        pltpu.VMEM((64, H), jnp.float32),
        pltpu.SemaphoreType.DMA,
    ]

    @functools.partial(pl.kernel, mesh=mesh, out_type=out_type,
                       scratch_types=scratch)
    def body(p1h, p2h, yh, a, b, myidx, rows, sem):
        c = lax.axis_index("c")
        s = lax.axis_index("s")
        base = (c * 16 + s) * 64
        pltpu.sync_copy(p1h.at[pl.ds(base, 64)], myidx)
        pltpu.async_copy(yh.at[myidx], rows, sem).wait()
        pltpu.sync_copy(rows, a.at[pl.ds(base, 64)])
        pltpu.sync_copy(p2h.at[pl.ds(base, 64)], myidx)
        pltpu.async_copy(yh.at[myidx], rows, sem).wait()
        pltpu.sync_copy(rows, b.at[pl.ds(base, 64)])

    return body(pos1, pos2, yr)


def _ffn_shared_body(x_ref, w1_ref, b1_ref, w2_ref, b2_ref, w3_ref, b3_ref,
                     a_ref, b2r_ref, out_ref):
    j = pl.program_id(1)
    xt = x_ref[...].astype(jnp.bfloat16)
    h1 = lax.dot_general(xt, w1_ref[...].astype(jnp.bfloat16),
                         (((1,), (1,)), ((), ())),
                         preferred_element_type=jnp.float32) + b1_ref[0]
    h3 = lax.dot_general(xt, w3_ref[...].astype(jnp.bfloat16),
                         (((1,), (1,)), ((), ())),
                         preferred_element_type=jnp.float32) + b3_ref[0]
    hh = (h1 * jax.nn.sigmoid(h1) * h3).astype(jnp.bfloat16)
    w2c = w2_ref[:, pl.ds(j * FFC, FFC)]
    y = lax.dot_general(hh, w2c.astype(jnp.bfloat16),
                        (((1,), (1,)), ((), ())),
                        preferred_element_type=jnp.float32)

    @pl.when(j == 0)
    def _():
        out_ref[...] = y + b2_ref[...][None, :] + a_ref[...] + b2r_ref[...]

    @pl.when(j != 0)
    def _():
        out_ref[...] = out_ref[...] + y


def _ffn_shared(x2d, sw1, sb1, sw2, sb2, sw3, sb3, a, b):
    return pl.pallas_call(
        _ffn_shared_body,
        grid=(ST, NFC),
        in_specs=[
            pl.BlockSpec((T, H), lambda i, j: (i, 0)),
            pl.BlockSpec((FFC, H), lambda i, j: (j, 0)),
            pl.BlockSpec((1, 1, FFC), lambda i, j: (j, 0, 0)),
            pl.BlockSpec((H, FF), lambda i, j: (0, 0)),
            pl.BlockSpec((H,), lambda i, j: (0,)),
            pl.BlockSpec((FFC, H), lambda i, j: (j, 0)),
            pl.BlockSpec((1, 1, FFC), lambda i, j: (j, 0, 0)),
            pl.BlockSpec((T, H), lambda i, j: (i, 0)),
            pl.BlockSpec((T, H), lambda i, j: (i, 0)),
        ],
        out_specs=pl.BlockSpec((T, H), lambda i, j: (i, 0)),
        out_shape=jax.ShapeDtypeStruct((S, H), jnp.float32),
        compiler_params=pltpu.CompilerParams(
            dimension_semantics=("arbitrary", "arbitrary"),
            vmem_limit_bytes=128 * 1024 * 1024,
        ),
    )(x2d, sw1, sb1, sw2, sb2, sw3, sb3, a, b)


def kernel(x, router_w, router_b, expert_biases, sw1, sb1, sw2, sb2, sw3, sb3,
           rw1, rb1, rw2, rb2, rw3, rb3):
    x2d = x.reshape(S, H)
    i1, i2, g1, g2 = _router(x2d, router_w, router_b, expert_biases)
    xs, gs, te, pos1, pos2 = _dispatch(i1, i2, g1, g2, x2d)
    gs3 = gs.reshape(NT, 1, T)
    rb1r = rb1.reshape(E, NFC, 1, FFC)
    rb3r = rb3.reshape(E, NFC, 1, FFC)
    rb2r = rb2.reshape(E, 1, H)
    yr = _ffn_routed(te, xs, gs3, rw1, rb1r, rw2, rb2r, rw3, rb3r)
    a, b = _combine_gather(pos1, pos2, yr)
    sb1r = sb1.reshape(NFC, 1, FFC)
    sb3r = sb3.reshape(NFC, 1, FFC)
    out = _ffn_shared(x2d, sw1, sb1r, sw2, sb2, sw3, sb3r, a, b)
    return out.reshape(1, S, H)


# speedup vs baseline: 1.1620x; 1.0465x over previous
import functools

import jax
import jax.numpy as jnp
from jax import lax
from jax.experimental import pallas as pl
from jax.experimental.pallas import tpu as pltpu
from jax.experimental.pallas import tpu_sc as plsc

S, H, FF, E = 2048, 1024, 2816, 8
T = 256
NT = 24
NPAD = NT * T
NP = 2 * S
NFC = 2
FFC = FF // NFC
ST = S // T


def _vgather16(v, idx):
    dn = lax.GatherDimensionNumbers(offset_dims=(), collapsed_slice_dims=(0,),
                                    start_index_map=(0,))
    return lax.gather(v, idx[:, None], dn, slice_sizes=(1,),
                      mode=lax.GatherScatterMode.PROMISE_IN_BOUNDS)


def _router_body(x_ref, rw_ref, rb_ref, eb_ref, i1_ref, i2_ref, g1_ref, g2_ref):
    x = x_ref[...]
    logits = lax.dot_general(x, rw_ref[...], (((1,), (1,)), ((), ())),
                             preferred_element_type=jnp.float32)
    logits = logits + rb_ref[...][None, :]
    lb = logits + eb_ref[...][None, :]
    eio = lax.broadcasted_iota(jnp.int32, lb.shape, 1)
    big = jnp.int32(1 << 30)
    m1 = jnp.max(lb, axis=-1, keepdims=True)
    i1 = jnp.min(jnp.where(lb == m1, eio, big), axis=-1, keepdims=True)
    lb2 = jnp.where(eio == i1, -jnp.inf, lb)
    m2 = jnp.max(lb2, axis=-1, keepdims=True)
    i2 = jnp.min(jnp.where(lb2 == m2, eio, big), axis=-1, keepdims=True)
    s1 = jnp.sum(jnp.where(eio == i1, logits, 0.0), axis=-1)
    s2 = jnp.sum(jnp.where(eio == i2, logits, 0.0), axis=-1)
    p1 = jax.nn.sigmoid(s1)
    p2 = jax.nn.sigmoid(s2)
    den = jnp.maximum(p1 + p2, 1e-12)
    i1_ref[...] = i1[:, 0]
    i2_ref[...] = i2[:, 0]
    g1_ref[...] = p1 / den
    g2_ref[...] = p2 / den


def _router(x2d, router_w, router_b, expert_biases):
    return pl.pallas_call(
        _router_body,
        out_shape=[
            jax.ShapeDtypeStruct((S,), jnp.int32),
            jax.ShapeDtypeStruct((S,), jnp.int32),
            jax.ShapeDtypeStruct((S,), jnp.float32),
            jax.ShapeDtypeStruct((S,), jnp.float32),
        ],
    )(x2d, router_w, router_b, expert_biases)


def _dispatch(i1, i2, g1, g2, x2d):
    mesh = plsc.VectorSubcoreMesh(core_axis_name="c", subcore_axis_name="s")
    out_type = [
        jax.ShapeDtypeStruct((NPAD, H), jnp.float32),
        jax.ShapeDtypeStruct((NPAD,), jnp.float32),
        jax.ShapeDtypeStruct((32,), jnp.int32),
        jax.ShapeDtypeStruct((S,), jnp.int32),
        jax.ShapeDtypeStruct((S,), jnp.int32),
    ]
    DS = NPAD // 16
    DP = NP // 16
    scratch = [
        pltpu.VMEM((NP,), jnp.int32),
        pltpu.VMEM((NP,), jnp.float32),
        pltpu.VMEM((NPAD,), jnp.int32),
        pltpu.VMEM((NPAD,), jnp.float32),
        pltpu.VMEM((NP,), jnp.int32),
        pltpu.VMEM((16,), jnp.int32),
        pltpu.VMEM((128,), jnp.int32),
        pltpu.VMEM((8, DS), jnp.int32),
        pltpu.VMEM((8, DS), jnp.float32),
        pltpu.VMEM((8, DP), jnp.int32),
        pltpu.VMEM((DS,), jnp.int32),
        pltpu.VMEM((DS,), jnp.float32),
        pltpu.VMEM((DP,), jnp.int32),
        pltpu.VMEM((32,), jnp.int32),
        pltpu.VMEM_SHARED((128,), jnp.int32),
        pltpu.VMEM_SHARED((8, NPAD), jnp.int32),
        pltpu.VMEM_SHARED((8, NPAD), jnp.float32),
        pltpu.VMEM_SHARED((8, NP), jnp.int32),
        pltpu.VMEM_SHARED((NPAD,), jnp.int32),
        pltpu.VMEM((64,), jnp.int32),
        pltpu.VMEM((64, H), jnp.float32),
        pltpu.SemaphoreType.DMA,
    ]

    @functools.partial(
        pl.kernel, mesh=mesh, out_type=out_type, scratch_types=scratch,
        compiler_params=pltpu.CompilerParams(needs_layout_passes=False))
    def body(i1h, i2h, g1h, g2h, xh, xs, gs, te, pos1, pos2,
             ep, gp, ptokp, gsortp, posbp, st16, cnta, mpt, mgs, mpb,
             mmpt, mmgs, mmpb, tebuf, sh_cnt, sh_ptokp, sh_gsortp, sh_posbp,
             shp, myidx, rows, sem):
        c = lax.axis_index("c")
        s = lax.axis_index("s")
        io16 = lax.iota(jnp.int32, 16)
        zi = jnp.zeros((16,), jnp.int32)
        zf = jnp.zeros((16,), jnp.float32)

        @pl.when(s < E)
        def _count():
            pltpu.sync_copy(i1h, ep.at[pl.ds(0, S)])
            pltpu.sync_copy(i2h, ep.at[pl.ds(S, S)])
            pltpu.sync_copy(g1h, gp.at[pl.ds(0, S)])
            pltpu.sync_copy(g2h, gp.at[pl.ds(S, S)])

            def init_body(i, carry):
                ptokp[pl.ds(i * 16, 16)] = zi
                gsortp[pl.ds(i * 16, 16)] = zf
                return carry

            lax.fori_loop(0, NPAD // 16, init_body, 0)

            def cbody(g, cnt):
                ev = ep[pl.ds(g * 16, 16)]
                return cnt + plsc.all_reduce_population_count(ev == s)

            cnt = lax.fori_loop(0, NP // 16, cbody, zi)
            st16[...] = cnt
            pltpu.sync_copy(st16, sh_cnt.at[pl.ds(s * 16, 16)])

        plsc.subcore_barrier()

        pltpu.sync_copy(sh_cnt, cnta)
        counts_v = zi
        for e in range(E):
            sp = cnta[pl.ds(e * 16, 16)]
            counts_v = jnp.where(io16 == e, sp, counts_v)
        ntiles_v = (counts_v + (T - 1)) // T
        tstart_v = plsc.cumsum(ntiles_v) - ntiles_v
        basevals = tstart_v * T

        @pl.when(s < E)
        def _scatter():
            mybase = _vgather16(basevals, jnp.full((16,), 0, jnp.int32) + s)

            def group2(g, carry):
                pidx, crun = carry
                ev = ep[pl.ds(g * 16, 16)]
                gv = gp[pl.ds(g * 16, 16)]
                m = ev == s
                mi = m.astype(jnp.int32)
                incl = plsc.cumsum(mi)
                posv = jnp.where(m, mybase + incl - 1 + crun, 0)
                tokv = pidx & (S - 1)
                plsc.store_scatter(ptokp, [posv], tokv, mask=m)
                plsc.store_scatter(gsortp, [posv], gv, mask=m)
                posbp[pl.ds(g * 16, 16)] = posv
                tote = plsc.cummax(lax.rev(incl, (0,)))
                return (pidx + 16, crun + tote)

            lax.fori_loop(0, NP // 16, group2, (io16, zi))
            pltpu.sync_copy(ptokp, sh_ptokp.at[s])
            pltpu.sync_copy(gsortp, sh_gsortp.at[s])
            pltpu.sync_copy(posbp, sh_posbp.at[s])

        @pl.when((s == 0) & (c == 0))
        def _te():
            for j in range(2):
                tv = io16 + j * 16
                ind = jnp.zeros((16,), jnp.int32)
                for e in range(E):
                    tse = _vgather16(tstart_v, jnp.full((16,), e, jnp.int32))
                    ind = ind + jnp.where(tv >= tse, 1, 0)
                tebuf[pl.ds(j * 16, 16)] = jnp.maximum(ind - 1, 0)
            pltpu.sync_copy(tebuf, te)

        plsc.subcore_barrier()

        DS = NPAD // 16
        DP = NP // 16
        pltpu.sync_copy(sh_ptokp.at[:, pl.ds(s * DS, DS)], mpt)
        pltpu.sync_copy(sh_gsortp.at[:, pl.ds(s * DS, DS)], mgs)
        pltpu.sync_copy(sh_posbp.at[:, pl.ds(s * DP, DP)], mpb)

        def merge_s(k, carry):
            acc_i = zi
            acc_f = zf
            for e in range(E):
                acc_i = acc_i + mpt[e, pl.ds(k * 16, 16)]
                acc_f = acc_f + mgs[e, pl.ds(k * 16, 16)]
            mmpt[pl.ds(k * 16, 16)] = acc_i
            mmgs[pl.ds(k * 16, 16)] = acc_f
            return carry

        lax.fori_loop(0, DS // 16, merge_s, 0)

        def merge_p(k, carry):
            acc_i = zi
            for e in range(E):
                acc_i = acc_i + mpb[e, pl.ds(k * 16, 16)]
            mmpb[pl.ds(k * 16, 16)] = acc_i
            return carry

        lax.fori_loop(0, DP // 16, merge_p, 0)

        pltpu.sync_copy(mmpt, shp.at[pl.ds(s * DS, DS)])

        @pl.when(c == 0)
        def _meta():
            pltpu.sync_copy(mmgs, gs.at[pl.ds(s * DS, DS)])

            @pl.when(s < 8)
            def _p1():
                pltpu.sync_copy(mmpb, pos1.at[pl.ds(s * DP, DP)])

            @pl.when(s >= 8)
            def _p2():
                pltpu.sync_copy(mmpb, pos2.at[pl.ds((s - 8) * DP, DP)])

        plsc.subcore_barrier()

        half = NPAD // 2
        per = half // 16
        for ch in range(per // 64):
            start = c * half + s * per + ch * 64
            pltpu.sync_copy(shp.at[pl.ds(start, 64)], myidx)
            pltpu.async_copy(xh.at[myidx], rows, sem).wait()
            pltpu.sync_copy(rows, xs.at[pl.ds(start, 64)])

    return body(i1, i2, g1, g2, x2d)


def _ffn_routed_body(te_ref, xs_ref, g_ref, w1_ref, b1_ref, w2_ref, b2_ref,
                     w3_ref, b3_ref, out_ref):
    i = pl.program_id(0)
    j = pl.program_id(1)
    jc = (i + j) % NFC
    xt = xs_ref[...].astype(jnp.bfloat16)
    gcol = g_ref[0, 0][:, None]
    h1 = lax.dot_general(xt, w1_ref[0].astype(jnp.bfloat16),
                         (((1,), (1,)), ((), ())),
                         preferred_element_type=jnp.float32) + b1_ref[0, 0]
    h3 = lax.dot_general(xt, w3_ref[0].astype(jnp.bfloat16),
                         (((1,), (1,)), ((), ())),
                         preferred_element_type=jnp.float32) + b3_ref[0, 0]
    hh = (h1 * jax.nn.sigmoid(h1) * h3).astype(jnp.bfloat16)
    w2c = w2_ref[0, :, pl.ds(jc * FFC, FFC)]
    y = lax.dot_general(hh, w2c.astype(jnp.bfloat16),
                        (((1,), (1,)), ((), ())),
                        preferred_element_type=jnp.float32)

    @pl.when(j == 0)
    def _():
        out_ref[...] = (y + b2_ref[0]) * gcol

    @pl.when(j != 0)
    def _():
        out_ref[...] = out_ref[...] + y * gcol


def _ffn_routed(te, xs, gs3, rw1, rb1, rw2, rb2, rw3, rb3):
    grid_spec = pltpu.PrefetchScalarGridSpec(
        num_scalar_prefetch=1,
        grid=(NT, NFC),
        in_specs=[
            pl.BlockSpec((T, H), lambda i, j, te: (i, 0)),
            pl.BlockSpec((1, 1, T), lambda i, j, te: (i, 0, 0)),
            pl.BlockSpec((1, FFC, H), lambda i, j, te: (te[i], (i + j) % NFC, 0)),
            pl.BlockSpec((1, 1, 1, FFC), lambda i, j, te: (te[i], (i + j) % NFC, 0, 0)),
            pl.BlockSpec((1, H, FF), lambda i, j, te: (te[i], 0, 0)),
            pl.BlockSpec((1, 1, H), lambda i, j, te: (te[i], 0, 0)),
            pl.BlockSpec((1, FFC, H), lambda i, j, te: (te[i], (i + j) % NFC, 0)),
            pl.BlockSpec((1, 1, 1, FFC), lambda i, j, te: (te[i], (i + j) % NFC, 0, 0)),
        ],
        out_specs=pl.BlockSpec((T, H), lambda i, j, te: (i, 0)),
    )
    return pl.pallas_call(
        _ffn_routed_body,
        grid_spec=grid_spec,
        out_shape=jax.ShapeDtypeStruct((NPAD, H), jnp.float32),
        compiler_params=pltpu.CompilerParams(
            dimension_semantics=("arbitrary", "arbitrary"),
            vmem_limit_bytes=128 * 1024 * 1024,
        ),
    )(te, xs, gs3, rw1, rb1, rw2, rb2, rw3, rb3)


def _combine_gather(pos1, pos2, yr):
    mesh = plsc.VectorSubcoreMesh(core_axis_name="c", subcore_axis_name="s")
    out_type = [
        jax.ShapeDtypeStruct((S, H), jnp.float32),
        jax.ShapeDtypeStruct((S, H), jnp.float32),
    ]
    scratch = [
        pltpu.VMEM((64,), jnp.int32),
        pltpu.VMEM((64, H), jnp.float32),
        pltpu.SemaphoreType.DMA,
    ]

    @functools.partial(pl.kernel, mesh=mesh, out_type=out_type,
                       scratch_types=scratch)
    def body(p1h, p2h, yh, a, b, myidx, rows, sem):
        c = lax.axis_index("c")
        s = lax.axis_index("s")
        base = (c * 16 + s) * 64
        pltpu.sync_copy(p1h.at[pl.ds(base, 64)], myidx)
        pltpu.async_copy(yh.at[myidx], rows, sem).wait()
        pltpu.sync_copy(rows, a.at[pl.ds(base, 64)])
        pltpu.sync_copy(p2h.at[pl.ds(base, 64)], myidx)
        pltpu.async_copy(yh.at[myidx], rows, sem).wait()
        pltpu.sync_copy(rows, b.at[pl.ds(base, 64)])

    return body(pos1, pos2, yr)


def _ffn_shared_body(x_ref, w1_ref, b1_ref, w2_ref, b2_ref, w3_ref, b3_ref,
                     a_ref, b2r_ref, out_ref):
    i = pl.program_id(0)
    j = pl.program_id(1)
    jc = (i + j) % NFC
    xt = x_ref[...].astype(jnp.bfloat16)
    h1 = lax.dot_general(xt, w1_ref[...].astype(jnp.bfloat16),
                         (((1,), (1,)), ((), ())),
                         preferred_element_type=jnp.float32) + b1_ref[0]
    h3 = lax.dot_general(xt, w3_ref[...].astype(jnp.bfloat16),
                         (((1,), (1,)), ((), ())),
                         preferred_element_type=jnp.float32) + b3_ref[0]
    hh = (h1 * jax.nn.sigmoid(h1) * h3).astype(jnp.bfloat16)
    w2c = w2_ref[:, pl.ds(jc * FFC, FFC)]
    y = lax.dot_general(hh, w2c.astype(jnp.bfloat16),
                        (((1,), (1,)), ((), ())),
                        preferred_element_type=jnp.float32)

    @pl.when(j == 0)
    def _():
        out_ref[...] = y + b2_ref[...][None, :] + a_ref[...] + b2r_ref[...]

    @pl.when(j != 0)
    def _():
        out_ref[...] = out_ref[...] + y


def _ffn_shared(x2d, sw1, sb1, sw2, sb2, sw3, sb3, a, b):
    return pl.pallas_call(
        _ffn_shared_body,
        grid=(ST, NFC),
        in_specs=[
            pl.BlockSpec((T, H), lambda i, j: (i, 0)),
            pl.BlockSpec((FFC, H), lambda i, j: ((i + j) % NFC, 0)),
            pl.BlockSpec((1, 1, FFC), lambda i, j: ((i + j) % NFC, 0, 0)),
            pl.BlockSpec((H, FF), lambda i, j: (0, 0)),
            pl.BlockSpec((H,), lambda i, j: (0,)),
            pl.BlockSpec((FFC, H), lambda i, j: ((i + j) % NFC, 0)),
            pl.BlockSpec((1, 1, FFC), lambda i, j: ((i + j) % NFC, 0, 0)),
            pl.BlockSpec((T, H), lambda i, j: (i, 0)),
            pl.BlockSpec((T, H), lambda i, j: (i, 0)),
        ],
        out_specs=pl.BlockSpec((T, H), lambda i, j: (i, 0)),
        out_shape=jax.ShapeDtypeStruct((S, H), jnp.float32),
        compiler_params=pltpu.CompilerParams(
            dimension_semantics=("arbitrary", "arbitrary"),
            vmem_limit_bytes=128 * 1024 * 1024,
        ),
    )(x2d, sw1, sb1, sw2, sb2, sw3, sb3, a, b)


def kernel(x, router_w, router_b, expert_biases, sw1, sb1, sw2, sb2, sw3, sb3,
           rw1, rb1, rw2, rb2, rw3, rb3):
    x2d = x.reshape(S, H)
    i1, i2, g1, g2 = _router(x2d, router_w, router_b, expert_biases)
    xs, gs, te, pos1, pos2 = _dispatch(i1, i2, g1, g2, x2d)
    gs3 = gs.reshape(NT, 1, T)
    rb1r = rb1.reshape(E, NFC, 1, FFC)
    rb3r = rb3.reshape(E, NFC, 1, FFC)
    rb2r = rb2.reshape(E, 1, H)
    yr = _ffn_routed(te, xs, gs3, rw1, rb1r, rw2, rb2r, rw3, rb3r)
    a, b = _combine_gather(pos1, pos2, yr)
    sb1r = sb1.reshape(NFC, 1, FFC)
    sb3r = sb3.reshape(NFC, 1, FFC)
    out = _ffn_shared(x2d, sw1, sb1r, sw2, sb2, sw3, sb3r, a, b)
    return out.reshape(1, S, H)


# speedup vs baseline: 1.2182x; 1.0484x over previous
import functools

import jax
import jax.numpy as jnp
from jax import lax
from jax.experimental import pallas as pl
from jax.experimental.pallas import tpu as pltpu
from jax.experimental.pallas import tpu_sc as plsc

S, H, FF, E = 2048, 1024, 2816, 8
T = 256
NT = 24
NPAD = NT * T
NP = 2 * S
NFC = 2
FFC = FF // NFC
ST = S // T


def _vgather16(v, idx):
    dn = lax.GatherDimensionNumbers(offset_dims=(), collapsed_slice_dims=(0,),
                                    start_index_map=(0,))
    return lax.gather(v, idx[:, None], dn, slice_sizes=(1,),
                      mode=lax.GatherScatterMode.PROMISE_IN_BOUNDS)


def _router_body(x_ref, rw_ref, rb_ref, eb_ref, i1_ref, i2_ref, g1_ref, g2_ref):
    x = x_ref[...]
    logits = lax.dot_general(x, rw_ref[...], (((1,), (1,)), ((), ())),
                             preferred_element_type=jnp.float32)
    logits = logits + rb_ref[...][None, :]
    lb = logits + eb_ref[...][None, :]
    eio = lax.broadcasted_iota(jnp.int32, lb.shape, 1)
    big = jnp.int32(1 << 30)
    m1 = jnp.max(lb, axis=-1, keepdims=True)
    i1 = jnp.min(jnp.where(lb == m1, eio, big), axis=-1, keepdims=True)
    lb2 = jnp.where(eio == i1, -jnp.inf, lb)
    m2 = jnp.max(lb2, axis=-1, keepdims=True)
    i2 = jnp.min(jnp.where(lb2 == m2, eio, big), axis=-1, keepdims=True)
    s1 = jnp.sum(jnp.where(eio == i1, logits, 0.0), axis=-1)
    s2 = jnp.sum(jnp.where(eio == i2, logits, 0.0), axis=-1)
    p1 = jax.nn.sigmoid(s1)
    p2 = jax.nn.sigmoid(s2)
    den = jnp.maximum(p1 + p2, 1e-12)
    i1_ref[...] = i1[:, 0]
    i2_ref[...] = i2[:, 0]
    g1_ref[...] = p1 / den
    g2_ref[...] = p2 / den


def _router(x2d, router_w, router_b, expert_biases):
    return pl.pallas_call(
        _router_body,
        out_shape=[
            jax.ShapeDtypeStruct((S,), jnp.int32),
            jax.ShapeDtypeStruct((S,), jnp.int32),
            jax.ShapeDtypeStruct((S,), jnp.float32),
            jax.ShapeDtypeStruct((S,), jnp.float32),
        ],
    )(x2d, router_w, router_b, expert_biases)


def _dispatch(i1, i2, g1, g2, x2d):
    mesh = plsc.VectorSubcoreMesh(core_axis_name="c", subcore_axis_name="s")
    out_type = [
        jax.ShapeDtypeStruct((NPAD, H), jnp.float32),
        jax.ShapeDtypeStruct((NPAD,), jnp.float32),
        jax.ShapeDtypeStruct((32,), jnp.int32),
        jax.ShapeDtypeStruct((S,), jnp.int32),
        jax.ShapeDtypeStruct((S,), jnp.int32),
    ]
    DS = NPAD // 16
    DP = NP // 16
    scratch = [
        pltpu.VMEM((NP,), jnp.int32),
        pltpu.VMEM((NP,), jnp.float32),
        pltpu.VMEM((NPAD,), jnp.int32),
        pltpu.VMEM((NPAD,), jnp.float32),
        pltpu.VMEM((NP,), jnp.int32),
        pltpu.VMEM((16,), jnp.int32),
        pltpu.VMEM((128,), jnp.int32),
        pltpu.VMEM((8, DS), jnp.int32),
        pltpu.VMEM((8, DS), jnp.float32),
        pltpu.VMEM((8, DP), jnp.int32),
        pltpu.VMEM((DS,), jnp.int32),
        pltpu.VMEM((DS,), jnp.float32),
        pltpu.VMEM((DP,), jnp.int32),
        pltpu.VMEM((32,), jnp.int32),
        pltpu.VMEM_SHARED((128,), jnp.int32),
        pltpu.VMEM_SHARED((8, NPAD), jnp.int32),
        pltpu.VMEM_SHARED((8, NPAD), jnp.float32),
        pltpu.VMEM_SHARED((8, NP), jnp.int32),
        pltpu.VMEM_SHARED((NPAD,), jnp.int32),
        pltpu.VMEM((64,), jnp.int32),
        pltpu.VMEM((64, H), jnp.float32),
        pltpu.SemaphoreType.DMA,
    ]

    @functools.partial(
        pl.kernel, mesh=mesh, out_type=out_type, scratch_types=scratch,
        compiler_params=pltpu.CompilerParams(needs_layout_passes=False))
    def body(i1h, i2h, g1h, g2h, xh, xs, gs, te, pos1, pos2,
             ep, gp, ptokp, gsortp, posbp, st16, cnta, mpt, mgs, mpb,
             mmpt, mmgs, mmpb, tebuf, sh_cnt, sh_ptokp, sh_gsortp, sh_posbp,
             shp, myidx, rows, sem):
        c = lax.axis_index("c")
        s = lax.axis_index("s")
        io16 = lax.iota(jnp.int32, 16)
        zi = jnp.zeros((16,), jnp.int32)
        zf = jnp.zeros((16,), jnp.float32)

        @pl.when(s < E)
        def _count():
            pltpu.sync_copy(i1h, ep.at[pl.ds(0, S)])
            pltpu.sync_copy(i2h, ep.at[pl.ds(S, S)])
            pltpu.sync_copy(g1h, gp.at[pl.ds(0, S)])
            pltpu.sync_copy(g2h, gp.at[pl.ds(S, S)])

            def init_body(i, carry):
                ptokp[pl.ds(i * 16, 16)] = zi
                gsortp[pl.ds(i * 16, 16)] = zf
                return carry

            lax.fori_loop(0, NPAD // 16, init_body, 0)

            def cbody(g, cnt):
                ev = ep[pl.ds(g * 16, 16)]
                return cnt + plsc.all_reduce_population_count(ev == s)

            cnt = lax.fori_loop(0, NP // 16, cbody, zi)
            st16[...] = cnt
            pltpu.sync_copy(st16, sh_cnt.at[pl.ds(s * 16, 16)])

        plsc.subcore_barrier()

        pltpu.sync_copy(sh_cnt, cnta)
        counts_v = zi
        for e in range(E):
            sp = cnta[pl.ds(e * 16, 16)]
            counts_v = jnp.where(io16 == e, sp, counts_v)
        ntiles_v = (counts_v + (T - 1)) // T
        tstart_v = plsc.cumsum(ntiles_v) - ntiles_v
        basevals = tstart_v * T

        @pl.when(s < E)
        def _scatter():
            mybase = _vgather16(basevals, jnp.full((16,), 0, jnp.int32) + s)

            def group2(g, carry):
                pidx, crun = carry
                ev = ep[pl.ds(g * 16, 16)]
                gv = gp[pl.ds(g * 16, 16)]
                m = ev == s
                mi = m.astype(jnp.int32)
                incl = plsc.cumsum(mi)
                posv = jnp.where(m, mybase + incl - 1 + crun, 0)
                tokv = pidx & (S - 1)
                plsc.store_scatter(ptokp, [posv], tokv, mask=m)
                plsc.store_scatter(gsortp, [posv], gv, mask=m)
                posbp[pl.ds(g * 16, 16)] = posv
                tote = plsc.cummax(lax.rev(incl, (0,)))
                return (pidx + 16, crun + tote)

            lax.fori_loop(0, NP // 16, group2, (io16, zi))
            pltpu.sync_copy(ptokp, sh_ptokp.at[s])
            pltpu.sync_copy(gsortp, sh_gsortp.at[s])
            pltpu.sync_copy(posbp, sh_posbp.at[s])

        @pl.when((s == 0) & (c == 0))
        def _te():
            for j in range(2):
                tv = io16 + j * 16
                ind = jnp.zeros((16,), jnp.int32)
                for e in range(E):
                    tse = _vgather16(tstart_v, jnp.full((16,), e, jnp.int32))
                    ind = ind + jnp.where(tv >= tse, 1, 0)
                tebuf[pl.ds(j * 16, 16)] = jnp.maximum(ind - 1, 0)
            pltpu.sync_copy(tebuf, te)

        plsc.subcore_barrier()

        DS = NPAD // 16
        DP = NP // 16
        pltpu.sync_copy(sh_ptokp.at[:, pl.ds(s * DS, DS)], mpt)
        pltpu.sync_copy(sh_gsortp.at[:, pl.ds(s * DS, DS)], mgs)
        pltpu.sync_copy(sh_posbp.at[:, pl.ds(s * DP, DP)], mpb)

        def merge_s(k, carry):
            acc_i = zi
            acc_f = zf
            for e in range(E):
                acc_i = acc_i + mpt[e, pl.ds(k * 16, 16)]
                acc_f = acc_f + mgs[e, pl.ds(k * 16, 16)]
            mmpt[pl.ds(k * 16, 16)] = acc_i
            mmgs[pl.ds(k * 16, 16)] = acc_f
            return carry

        lax.fori_loop(0, DS // 16, merge_s, 0)

        def merge_p(k, carry):
            acc_i = zi
            for e in range(E):
                acc_i = acc_i + mpb[e, pl.ds(k * 16, 16)]
            mmpb[pl.ds(k * 16, 16)] = acc_i
            return carry

        lax.fori_loop(0, DP // 16, merge_p, 0)

        pltpu.sync_copy(mmpt, shp.at[pl.ds(s * DS, DS)])

        @pl.when(c == 0)
        def _meta():
            pltpu.sync_copy(mmgs, gs.at[pl.ds(s * DS, DS)])

            @pl.when(s < 8)
            def _p1():
                pltpu.sync_copy(mmpb, pos1.at[pl.ds(s * DP, DP)])

            @pl.when(s >= 8)
            def _p2():
                pltpu.sync_copy(mmpb, pos2.at[pl.ds((s - 8) * DP, DP)])

        plsc.subcore_barrier()

        half = NPAD // 2
        per = half // 16
        for ch in range(per // 64):
            start = c * half + s * per + ch * 64
            pltpu.sync_copy(shp.at[pl.ds(start, 64)], myidx)
            pltpu.async_copy(xh.at[myidx], rows, sem).wait()
            pltpu.sync_copy(rows, xs.at[pl.ds(start, 64)])

    return body(i1, i2, g1, g2, x2d)


def _ffn_routed_body(te_ref, xs_ref, g_ref, w1_ref, b1_ref, w2_ref, b2_ref,
                     w3_ref, b3_ref, out_ref):
    i = pl.program_id(0)
    j = pl.program_id(1)
    jc = (i + j) % NFC
    xt = xs_ref[...].astype(jnp.bfloat16)
    gcol = g_ref[0, 0][:, None]
    h1 = lax.dot_general(xt, w1_ref[0].astype(jnp.bfloat16),
                         (((1,), (1,)), ((), ())),
                         preferred_element_type=jnp.float32) + b1_ref[0, 0]
    h3 = lax.dot_general(xt, w3_ref[0].astype(jnp.bfloat16),
                         (((1,), (1,)), ((), ())),
                         preferred_element_type=jnp.float32) + b3_ref[0, 0]
    hh = (h1 * jax.nn.sigmoid(h1) * h3).astype(jnp.bfloat16)
    w2c = w2_ref[0, :, pl.ds(jc * FFC, FFC)]
    y = lax.dot_general(hh, w2c.astype(jnp.bfloat16),
                        (((1,), (1,)), ((), ())),
                        preferred_element_type=jnp.float32)

    @pl.when(j == 0)
    def _():
        out_ref[...] = (y + b2_ref[0]) * gcol

    @pl.when(j != 0)
    def _():
        out_ref[...] = out_ref[...] + y * gcol


def _ffn_routed(te, xs, gs3, rw1, rb1, rw2, rb2, rw3, rb3):
    grid_spec = pltpu.PrefetchScalarGridSpec(
        num_scalar_prefetch=1,
        grid=(NT, NFC),
        in_specs=[
            pl.BlockSpec((T, H), lambda i, j, te: (i, 0)),
            pl.BlockSpec((1, 1, T), lambda i, j, te: (i, 0, 0)),
            pl.BlockSpec((1, FFC, H), lambda i, j, te: (te[i], (i + j) % NFC, 0)),
            pl.BlockSpec((1, 1, 1, FFC), lambda i, j, te: (te[i], (i + j) % NFC, 0, 0)),
            pl.BlockSpec((1, H, FF), lambda i, j, te: (te[i], 0, 0)),
            pl.BlockSpec((1, 1, H), lambda i, j, te: (te[i], 0, 0)),
            pl.BlockSpec((1, FFC, H), lambda i, j, te: (te[i], (i + j) % NFC, 0)),
            pl.BlockSpec((1, 1, 1, FFC), lambda i, j, te: (te[i], (i + j) % NFC, 0, 0)),
        ],
        out_specs=pl.BlockSpec((T, H), lambda i, j, te: (i, 0)),
    )
    return pl.pallas_call(
        _ffn_routed_body,
        grid_spec=grid_spec,
        out_shape=jax.ShapeDtypeStruct((NPAD, H), jnp.float32),
        compiler_params=pltpu.CompilerParams(
            dimension_semantics=("arbitrary", "arbitrary"),
            vmem_limit_bytes=128 * 1024 * 1024,
        ),
    )(te, xs, gs3, rw1, rb1, rw2, rb2, rw3, rb3)


def _combine_final(pos1, pos2, yr, ys):
    mesh = plsc.VectorSubcoreMesh(core_axis_name="c", subcore_axis_name="s")
    out_type = [jax.ShapeDtypeStruct((S, H), jnp.float32)]
    CH = 32
    scratch = [
        pltpu.VMEM((CH,), jnp.int32),
        pltpu.VMEM((CH, H), jnp.float32),
        pltpu.VMEM((CH, H), jnp.float32),
        pltpu.VMEM((CH, H), jnp.float32),
        pltpu.SemaphoreType.DMA,
    ]

    @functools.partial(pl.kernel, mesh=mesh, out_type=out_type,
                       scratch_types=scratch)
    def body(p1h, p2h, yh, ysh, out, myidx, r1, r2, r3, sem):
        c = lax.axis_index("c")
        s = lax.axis_index("s")
        wid = c * 16 + s
        for ch in range(64 // CH):
            base = wid * 64 + ch * CH
            pltpu.sync_copy(p1h.at[pl.ds(base, CH)], myidx)
            pltpu.async_copy(yh.at[myidx], r1, sem).wait()
            pltpu.sync_copy(p2h.at[pl.ds(base, CH)], myidx)
            pltpu.async_copy(yh.at[myidx], r2, sem).wait()
            pltpu.sync_copy(ysh.at[pl.ds(base, CH)], r3)

            def addb(k, carry):
                for u in range(8):
                    q = k * 8 + u
                    row = q >> 6
                    col = (q & 63) * 16
                    r1[row, pl.ds(col, 16)] = (r1[row, pl.ds(col, 16)]
                                               + r2[row, pl.ds(col, 16)]
                                               + r3[row, pl.ds(col, 16)])
                return carry

            lax.fori_loop(0, CH * (H // 16) // 8, addb, 0)
            pltpu.sync_copy(r1, out.at[pl.ds(base, CH)])

    return body(pos1, pos2, yr, ys)[0]


def _ffn_shared_body(x_ref, w1_ref, b1_ref, w2_ref, b2_ref, w3_ref, b3_ref,
                     out_ref):
    i = pl.program_id(0)
    j = pl.program_id(1)
    jc = (i + j) % NFC
    xt = x_ref[...].astype(jnp.bfloat16)
    h1 = lax.dot_general(xt, w1_ref[...].astype(jnp.bfloat16),
                         (((1,), (1,)), ((), ())),
                         preferred_element_type=jnp.float32) + b1_ref[0]
    h3 = lax.dot_general(xt, w3_ref[...].astype(jnp.bfloat16),
                         (((1,), (1,)), ((), ())),
                         preferred_element_type=jnp.float32) + b3_ref[0]
    hh = (h1 * jax.nn.sigmoid(h1) * h3).astype(jnp.bfloat16)
    w2c = w2_ref[:, pl.ds(jc * FFC, FFC)]
    y = lax.dot_general(hh, w2c.astype(jnp.bfloat16),
                        (((1,), (1,)), ((), ())),
                        preferred_element_type=jnp.float32)

    @pl.when(j == 0)
    def _():
        out_ref[...] = y + b2_ref[...][None, :]

    @pl.when(j != 0)
    def _():
        out_ref[...] = out_ref[...] + y


def _ffn_shared(x2d, sw1, sb1, sw2, sb2, sw3, sb3):
    return pl.pallas_call(
        _ffn_shared_body,
        grid=(ST, NFC),
        in_specs=[
            pl.BlockSpec((T, H), lambda i, j: (i, 0)),
            pl.BlockSpec((FFC, H), lambda i, j: ((i + j) % NFC, 0)),
            pl.BlockSpec((1, 1, FFC), lambda i, j: ((i + j) % NFC, 0, 0)),
            pl.BlockSpec((H, FF), lambda i, j: (0, 0)),
            pl.BlockSpec((H,), lambda i, j: (0,)),
            pl.BlockSpec((FFC, H), lambda i, j: ((i + j) % NFC, 0)),
            pl.BlockSpec((1, 1, FFC), lambda i, j: ((i + j) % NFC, 0, 0)),
        ],
        out_specs=pl.BlockSpec((T, H), lambda i, j: (i, 0)),
        out_shape=jax.ShapeDtypeStruct((S, H), jnp.float32),
        compiler_params=pltpu.CompilerParams(
            dimension_semantics=("arbitrary", "arbitrary"),
            vmem_limit_bytes=128 * 1024 * 1024,
        ),
    )(x2d, sw1, sb1, sw2, sb2, sw3, sb3)


def kernel(x, router_w, router_b, expert_biases, sw1, sb1, sw2, sb2, sw3, sb3,
           rw1, rb1, rw2, rb2, rw3, rb3):
    x2d = x.reshape(S, H)
    i1, i2, g1, g2 = _router(x2d, router_w, router_b, expert_biases)
    xs, gs, te, pos1, pos2 = _dispatch(i1, i2, g1, g2, x2d)
    gs3 = gs.reshape(NT, 1, T)
    rb1r = rb1.reshape(E, NFC, 1, FFC)
    rb3r = rb3.reshape(E, NFC, 1, FFC)
    rb2r = rb2.reshape(E, 1, H)
    yr = _ffn_routed(te, xs, gs3, rw1, rb1r, rw2, rb2r, rw3, rb3r)
    sb1r = sb1.reshape(NFC, 1, FFC)
    sb3r = sb3.reshape(NFC, 1, FFC)
    ys = _ffn_shared(x2d, sw1, sb1r, sw2, sb2, sw3, sb3r)
    out = _combine_final(pos1, pos2, yr, ys)
    return out.reshape(1, S, H)


# speedup vs baseline: 1.2511x; 1.0270x over previous
import functools

import jax
import jax.numpy as jnp
from jax import lax
from jax.experimental import pallas as pl
from jax.experimental.pallas import tpu as pltpu
from jax.experimental.pallas import tpu_sc as plsc

S, H, FF, E = 2048, 1024, 2816, 8
T = 256
NT = 24
NPAD = NT * T
NP = 2 * S
NFC = 2
FFC = FF // NFC
ST = S // T


def _vgather16(v, idx):
    dn = lax.GatherDimensionNumbers(offset_dims=(), collapsed_slice_dims=(0,),
                                    start_index_map=(0,))
    return lax.gather(v, idx[:, None], dn, slice_sizes=(1,),
                      mode=lax.GatherScatterMode.PROMISE_IN_BOUNDS)


def _router_body(x_ref, rw_ref, rb_ref, eb_ref, i1_ref, i2_ref, g1_ref, g2_ref):
    x = x_ref[...]
    logits = lax.dot_general(x, rw_ref[...], (((1,), (1,)), ((), ())),
                             preferred_element_type=jnp.float32)
    logits = logits + rb_ref[...][None, :]
    lb = logits + eb_ref[...][None, :]
    eio = lax.broadcasted_iota(jnp.int32, lb.shape, 1)
    big = jnp.int32(1 << 30)
    m1 = jnp.max(lb, axis=-1, keepdims=True)
    i1 = jnp.min(jnp.where(lb == m1, eio, big), axis=-1, keepdims=True)
    lb2 = jnp.where(eio == i1, -jnp.inf, lb)
    m2 = jnp.max(lb2, axis=-1, keepdims=True)
    i2 = jnp.min(jnp.where(lb2 == m2, eio, big), axis=-1, keepdims=True)
    s1 = jnp.sum(jnp.where(eio == i1, logits, 0.0), axis=-1)
    s2 = jnp.sum(jnp.where(eio == i2, logits, 0.0), axis=-1)
    p1 = jax.nn.sigmoid(s1)
    p2 = jax.nn.sigmoid(s2)
    den = jnp.maximum(p1 + p2, 1e-12)
    i1_ref[...] = i1[:, 0]
    i2_ref[...] = i2[:, 0]
    g1_ref[...] = p1 / den
    g2_ref[...] = p2 / den


def _router(x2d, router_w, router_b, expert_biases):
    return pl.pallas_call(
        _router_body,
        out_shape=[
            jax.ShapeDtypeStruct((S,), jnp.int32),
            jax.ShapeDtypeStruct((S,), jnp.int32),
            jax.ShapeDtypeStruct((S,), jnp.float32),
            jax.ShapeDtypeStruct((S,), jnp.float32),
        ],
    )(x2d, router_w, router_b, expert_biases)


def _dispatch(i1, i2, g1, g2, x2d):
    mesh = plsc.VectorSubcoreMesh(core_axis_name="c", subcore_axis_name="s")
    out_type = [
        jax.ShapeDtypeStruct((NPAD, H), jnp.float32),
        jax.ShapeDtypeStruct((NPAD,), jnp.float32),
        jax.ShapeDtypeStruct((32,), jnp.int32),
        jax.ShapeDtypeStruct((S,), jnp.int32),
        jax.ShapeDtypeStruct((S,), jnp.int32),
    ]
    DS = NPAD // 16
    DP = NP // 16
    scratch = [
        pltpu.VMEM((NP,), jnp.int32),
        pltpu.VMEM((NP,), jnp.float32),
        pltpu.VMEM((NPAD,), jnp.int32),
        pltpu.VMEM((NPAD,), jnp.float32),
        pltpu.VMEM((NP,), jnp.int32),
        pltpu.VMEM((16,), jnp.int32),
        pltpu.VMEM((128,), jnp.int32),
        pltpu.VMEM((8, DS), jnp.int32),
        pltpu.VMEM((8, DS), jnp.float32),
        pltpu.VMEM((8, DP), jnp.int32),
        pltpu.VMEM((DS,), jnp.int32),
        pltpu.VMEM((DS,), jnp.float32),
        pltpu.VMEM((DP,), jnp.int32),
        pltpu.VMEM((32,), jnp.int32),
        pltpu.VMEM_SHARED((128,), jnp.int32),
        pltpu.VMEM_SHARED((8, NPAD), jnp.int32),
        pltpu.VMEM_SHARED((8, NPAD), jnp.float32),
        pltpu.VMEM_SHARED((8, NP), jnp.int32),
        pltpu.VMEM_SHARED((NPAD,), jnp.int32),
        pltpu.VMEM((64,), jnp.int32),
        pltpu.VMEM((64, H), jnp.float32),
        pltpu.SemaphoreType.DMA,
    ]

    @functools.partial(
        pl.kernel, mesh=mesh, out_type=out_type, scratch_types=scratch,
        compiler_params=pltpu.CompilerParams(needs_layout_passes=False))
    def body(i1h, i2h, g1h, g2h, xh, xs, gs, te, pos1, pos2,
             ep, gp, ptokp, gsortp, posbp, st16, cnta, mpt, mgs, mpb,
             mmpt, mmgs, mmpb, tebuf, sh_cnt, sh_ptokp, sh_gsortp, sh_posbp,
             shp, myidx, rows, sem):
        c = lax.axis_index("c")
        s = lax.axis_index("s")
        io16 = lax.iota(jnp.int32, 16)
        zi = jnp.zeros((16,), jnp.int32)
        zf = jnp.zeros((16,), jnp.float32)

        @pl.when(s < E)
        def _count():
            pltpu.sync_copy(i1h, ep.at[pl.ds(0, S)])
            pltpu.sync_copy(i2h, ep.at[pl.ds(S, S)])
            pltpu.sync_copy(g1h, gp.at[pl.ds(0, S)])
            pltpu.sync_copy(g2h, gp.at[pl.ds(S, S)])

            def init_body(i, carry):
                ptokp[pl.ds(i * 16, 16)] = zi
                gsortp[pl.ds(i * 16, 16)] = zf
                return carry

            lax.fori_loop(0, NPAD // 16, init_body, 0)

            def cbody(g, cnt):
                ev = ep[pl.ds(g * 16, 16)]
                return cnt + plsc.all_reduce_population_count(ev == s)

            cnt = lax.fori_loop(0, NP // 16, cbody, zi)
            st16[...] = cnt
            pltpu.sync_copy(st16, sh_cnt.at[pl.ds(s * 16, 16)])

        plsc.subcore_barrier()

        pltpu.sync_copy(sh_cnt, cnta)
        counts_v = zi
        for e in range(E):
            sp = cnta[pl.ds(e * 16, 16)]
            counts_v = jnp.where(io16 == e, sp, counts_v)
        ntiles_v = (counts_v + (T - 1)) // T
        tstart_v = plsc.cumsum(ntiles_v) - ntiles_v
        basevals = tstart_v * T

        @pl.when(s < E)
        def _scatter():
            mybase = _vgather16(basevals, jnp.full((16,), 0, jnp.int32) + s)

            def group2(g, carry):
                pidx, crun = carry
                ev = ep[pl.ds(g * 16, 16)]
                gv = gp[pl.ds(g * 16, 16)]
                m = ev == s
                mi = m.astype(jnp.int32)
                incl = plsc.cumsum(mi)
                posv = jnp.where(m, mybase + incl - 1 + crun, 0)
                tokv = pidx & (S - 1)
                plsc.store_scatter(ptokp, [posv], tokv, mask=m)
                plsc.store_scatter(gsortp, [posv], gv, mask=m)
                posbp[pl.ds(g * 16, 16)] = posv
                tote = plsc.cummax(lax.rev(incl, (0,)))
                return (pidx + 16, crun + tote)

            lax.fori_loop(0, NP // 16, group2, (io16, zi))
            pltpu.sync_copy(ptokp, sh_ptokp.at[s])
            pltpu.sync_copy(gsortp, sh_gsortp.at[s])
            pltpu.sync_copy(posbp, sh_posbp.at[s])

        @pl.when((s == 0) & (c == 0))
        def _te():
            for j in range(2):
                tv = io16 + j * 16
                ind = jnp.zeros((16,), jnp.int32)
                for e in range(E):
                    tse = _vgather16(tstart_v, jnp.full((16,), e, jnp.int32))
                    ind = ind + jnp.where(tv >= tse, 1, 0)
                tebuf[pl.ds(j * 16, 16)] = jnp.maximum(ind - 1, 0)
            pltpu.sync_copy(tebuf, te)

        plsc.subcore_barrier()

        DS = NPAD // 16
        DP = NP // 16
        pltpu.sync_copy(sh_ptokp.at[:, pl.ds(s * DS, DS)], mpt)
        pltpu.sync_copy(sh_gsortp.at[:, pl.ds(s * DS, DS)], mgs)
        pltpu.sync_copy(sh_posbp.at[:, pl.ds(s * DP, DP)], mpb)

        def merge_s(k, carry):
            acc_i = zi
            acc_f = zf
            for e in range(E):
                acc_i = acc_i + mpt[e, pl.ds(k * 16, 16)]
                acc_f = acc_f + mgs[e, pl.ds(k * 16, 16)]
            mmpt[pl.ds(k * 16, 16)] = acc_i
            mmgs[pl.ds(k * 16, 16)] = acc_f
            return carry

        lax.fori_loop(0, DS // 16, merge_s, 0)

        def merge_p(k, carry):
            acc_i = zi
            for e in range(E):
                acc_i = acc_i + mpb[e, pl.ds(k * 16, 16)]
            mmpb[pl.ds(k * 16, 16)] = acc_i
            return carry

        lax.fori_loop(0, DP // 16, merge_p, 0)

        pltpu.sync_copy(mmpt, shp.at[pl.ds(s * DS, DS)])

        @pl.when(c == 0)
        def _meta():
            pltpu.sync_copy(mmgs, gs.at[pl.ds(s * DS, DS)])

            @pl.when(s < 8)
            def _p1():
                pltpu.sync_copy(mmpb, pos1.at[pl.ds(s * DP, DP)])

            @pl.when(s >= 8)
            def _p2():
                pltpu.sync_copy(mmpb, pos2.at[pl.ds((s - 8) * DP, DP)])

        plsc.subcore_barrier()

        half = NPAD // 2
        per = half // 16
        for ch in range(per // 64):
            start = c * half + s * per + ch * 64
            pltpu.sync_copy(shp.at[pl.ds(start, 64)], myidx)
            pltpu.async_copy(xh.at[myidx], rows, sem).wait()
            pltpu.sync_copy(rows, xs.at[pl.ds(start, 64)])

    return body(i1, i2, g1, g2, x2d)


def _ffn_routed_body(te_ref, xs_ref, g_ref, w1_ref, b1_ref, w2_ref, b2_ref,
                     w3_ref, b3_ref, out_ref):
    i = pl.program_id(0)
    j = pl.program_id(1)
    jc = (i + j) % NFC
    gcol = g_ref[0, 0][:, None]
    live = jnp.sum(g_ref[0, 0]) > 0.0

    @pl.when(live)
    def _compute():
        xt = xs_ref[...].astype(jnp.bfloat16)
        h1 = lax.dot_general(xt, w1_ref[0].astype(jnp.bfloat16),
                             (((1,), (1,)), ((), ())),
                             preferred_element_type=jnp.float32) + b1_ref[0, 0]
        h3 = lax.dot_general(xt, w3_ref[0].astype(jnp.bfloat16),
                             (((1,), (1,)), ((), ())),
                             preferred_element_type=jnp.float32) + b3_ref[0, 0]
        hh = (h1 * jax.nn.sigmoid(h1) * h3).astype(jnp.bfloat16)
        w2c = w2_ref[0, :, pl.ds(jc * FFC, FFC)]
        y = lax.dot_general(hh, w2c.astype(jnp.bfloat16),
                            (((1,), (1,)), ((), ())),
                            preferred_element_type=jnp.float32)

        @pl.when(j == 0)
        def _():
            out_ref[...] = (y + b2_ref[0]) * gcol

        @pl.when(j != 0)
        def _():
            out_ref[...] = out_ref[...] + y * gcol

    @pl.when(jnp.logical_not(live) & (j == 0))
    def _dead():
        out_ref[...] = jnp.zeros((T, H), jnp.float32)


def _ffn_routed(te, xs, gs3, rw1, rb1, rw2, rb2, rw3, rb3):
    grid_spec = pltpu.PrefetchScalarGridSpec(
        num_scalar_prefetch=1,
        grid=(NT, NFC),
        in_specs=[
            pl.BlockSpec((T, H), lambda i, j, te: (i, 0)),
            pl.BlockSpec((1, 1, T), lambda i, j, te: (i, 0, 0)),
            pl.BlockSpec((1, FFC, H), lambda i, j, te: (te[i], (i + j) % NFC, 0)),
            pl.BlockSpec((1, 1, 1, FFC), lambda i, j, te: (te[i], (i + j) % NFC, 0, 0)),
            pl.BlockSpec((1, H, FF), lambda i, j, te: (te[i], 0, 0)),
            pl.BlockSpec((1, 1, H), lambda i, j, te: (te[i], 0, 0)),
            pl.BlockSpec((1, FFC, H), lambda i, j, te: (te[i], (i + j) % NFC, 0)),
            pl.BlockSpec((1, 1, 1, FFC), lambda i, j, te: (te[i], (i + j) % NFC, 0, 0)),
        ],
        out_specs=pl.BlockSpec((T, H), lambda i, j, te: (i, 0)),
    )
    return pl.pallas_call(
        _ffn_routed_body,
        grid_spec=grid_spec,
        out_shape=jax.ShapeDtypeStruct((NPAD, H), jnp.float32),
        compiler_params=pltpu.CompilerParams(
            dimension_semantics=("arbitrary", "arbitrary"),
            vmem_limit_bytes=128 * 1024 * 1024,
        ),
    )(te, xs, gs3, rw1, rb1, rw2, rb2, rw3, rb3)


def _combine_final(pos1, pos2, yr, ys):
    mesh = plsc.VectorSubcoreMesh(core_axis_name="c", subcore_axis_name="s")
    out_type = [jax.ShapeDtypeStruct((S, H), jnp.float32)]
    CH = 32
    scratch = [
        pltpu.VMEM((CH,), jnp.int32),
        pltpu.VMEM((CH, H), jnp.float32),
        pltpu.VMEM((CH, H), jnp.float32),
        pltpu.VMEM((CH, H), jnp.float32),
        pltpu.SemaphoreType.DMA,
    ]

    @functools.partial(pl.kernel, mesh=mesh, out_type=out_type,
                       scratch_types=scratch)
    def body(p1h, p2h, yh, ysh, out, myidx, r1, r2, r3, sem):
        c = lax.axis_index("c")
        s = lax.axis_index("s")
        wid = c * 16 + s
        for ch in range(64 // CH):
            base = wid * 64 + ch * CH
            pltpu.sync_copy(p1h.at[pl.ds(base, CH)], myidx)
            pltpu.async_copy(yh.at[myidx], r1, sem).wait()
            pltpu.sync_copy(p2h.at[pl.ds(base, CH)], myidx)
            pltpu.async_copy(yh.at[myidx], r2, sem).wait()
            pltpu.sync_copy(ysh.at[pl.ds(base, CH)], r3)

            def addb(k, carry):
                for u in range(8):
                    q = k * 8 + u
                    row = q >> 6
                    col = (q & 63) * 16
                    r1[row, pl.ds(col, 16)] = (r1[row, pl.ds(col, 16)]
                                               + r2[row, pl.ds(col, 16)]
                                               + r3[row, pl.ds(col, 16)])
                return carry

            lax.fori_loop(0, CH * (H // 16) // 8, addb, 0)
            pltpu.sync_copy(r1, out.at[pl.ds(base, CH)])

    return body(pos1, pos2, yr, ys)[0]


def _ffn_shared_body(x_ref, w1_ref, b1_ref, w2_ref, b2_ref, w3_ref, b3_ref,
                     out_ref):
    i = pl.program_id(0)
    j = pl.program_id(1)
    jc = (i + j) % NFC
    xt = x_ref[...].astype(jnp.bfloat16)
    h1 = lax.dot_general(xt, w1_ref[...].astype(jnp.bfloat16),
                         (((1,), (1,)), ((), ())),
                         preferred_element_type=jnp.float32) + b1_ref[0]
    h3 = lax.dot_general(xt, w3_ref[...].astype(jnp.bfloat16),
                         (((1,), (1,)), ((), ())),
                         preferred_element_type=jnp.float32) + b3_ref[0]
    hh = (h1 * jax.nn.sigmoid(h1) * h3).astype(jnp.bfloat16)
    w2c = w2_ref[:, pl.ds(jc * FFC, FFC)]
    y = lax.dot_general(hh, w2c.astype(jnp.bfloat16),
                        (((1,), (1,)), ((), ())),
                        preferred_element_type=jnp.float32)

    @pl.when(j == 0)
    def _():
        out_ref[...] = y + b2_ref[...][None, :]

    @pl.when(j != 0)
    def _():
        out_ref[...] = out_ref[...] + y


def _ffn_shared(x2d, sw1, sb1, sw2, sb2, sw3, sb3):
    return pl.pallas_call(
        _ffn_shared_body,
        grid=(ST, NFC),
        in_specs=[
            pl.BlockSpec((T, H), lambda i, j: (i, 0)),
            pl.BlockSpec((FFC, H), lambda i, j: ((i + j) % NFC, 0)),
            pl.BlockSpec((1, 1, FFC), lambda i, j: ((i + j) % NFC, 0, 0)),
            pl.BlockSpec((H, FF), lambda i, j: (0, 0)),
            pl.BlockSpec((H,), lambda i, j: (0,)),
            pl.BlockSpec((FFC, H), lambda i, j: ((i + j) % NFC, 0)),
            pl.BlockSpec((1, 1, FFC), lambda i, j: ((i + j) % NFC, 0, 0)),
        ],
        out_specs=pl.BlockSpec((T, H), lambda i, j: (i, 0)),
        out_shape=jax.ShapeDtypeStruct((S, H), jnp.float32),
        compiler_params=pltpu.CompilerParams(
            dimension_semantics=("arbitrary", "arbitrary"),
            vmem_limit_bytes=128 * 1024 * 1024,
        ),
    )(x2d, sw1, sb1, sw2, sb2, sw3, sb3)


def kernel(x, router_w, router_b, expert_biases, sw1, sb1, sw2, sb2, sw3, sb3,
           rw1, rb1, rw2, rb2, rw3, rb3):
    x2d = x.reshape(S, H)
    i1, i2, g1, g2 = _router(x2d, router_w, router_b, expert_biases)
    xs, gs, te, pos1, pos2 = _dispatch(i1, i2, g1, g2, x2d)
    gs3 = gs.reshape(NT, 1, T)
    rb1r = rb1.reshape(E, NFC, 1, FFC)
    rb3r = rb3.reshape(E, NFC, 1, FFC)
    rb2r = rb2.reshape(E, 1, H)
    yr = _ffn_routed(te, xs, gs3, rw1, rb1r, rw2, rb2r, rw3, rb3r)
    sb1r = sb1.reshape(NFC, 1, FFC)
    sb3r = sb3.reshape(NFC, 1, FFC)
    ys = _ffn_shared(x2d, sw1, sb1r, sw2, sb2, sw3, sb3r)
    out = _combine_final(pos1, pos2, yr, ys)
    return out.reshape(1, S, H)
